# baseline (speedup 1.0000x reference)
import functools

import jax
import jax.numpy as jnp
from jax import lax
from jax.experimental import pallas as pl
from jax.experimental.pallas import tpu as pltpu
from jax.experimental.pallas import tpu_sc as plsc

_INFO = plsc.get_sparse_core_info()
_NC, _NS = _INFO.num_cores, _INFO.num_subcores
_NW = _NC * _NS


@functools.lru_cache(maxsize=None)
def _make_gather(V, D, B, S):
    assert B % _NW == 0
    b_per_w = B // _NW
    ch = 80
    assert b_per_w % ch == 0
    n_ch = b_per_w // ch
    nslab = S // 2
    rows_per_slab = b_per_w // nslab
    nb_w = b_per_w // S
    mesh = plsc.VectorSubcoreMesh(core_axis_name="c", subcore_axis_name="s")

    @functools.partial(
        pl.kernel,
        mesh=mesh,
        compiler_params=pltpu.CompilerParams(
            use_tc_tiling_on_sc=False, needs_layout_passes=False
        ),
        out_type=jax.ShapeDtypeStruct((nslab, _NW * rows_per_slab, D), jnp.float32),
        scratch_types=[
            pltpu.VMEM((b_per_w,), jnp.int32),
            pltpu.VMEM((b_per_w,), jnp.int32),
            pltpu.VMEM((b_per_w, D), jnp.float32),
            pltpu.SemaphoreType.DMA,
            pltpu.SemaphoreType.DMA,
        ],
    )
    def gather_kernel(idx_hbm, table_hbm, out_hbm, idx_v, idx_p, rows_v, sem, osem):
        wid = lax.axis_index("s") * _NC + lax.axis_index("c")
        base = wid * b_per_w
        pltpu.sync_copy(idx_hbm.at[pl.ds(base, b_per_w)], idx_v)
        lane = lax.iota(jnp.int32, 16)
        for j in range(b_per_w // 16):
            r = (16 * j) // rows_per_slab
            rem = (16 * j - r * rows_per_slab) + lane
            b_l = rem >> 1
            sp = rem & 1
            q = b_l * S + 2 * r + sp
            idx_p[pl.ds(j * 16, 16)] = plsc.load_gather(idx_v, [q])
        gathers = []
        for j in range(n_ch):
            gathers.append(
                pltpu.async_copy(
                    table_hbm.at[idx_p.at[pl.ds(j * ch, ch)]],
                    rows_v.at[pl.ds(j * ch, ch)],
                    sem,
                )
            )
        for c in gathers:
            c.wait()
        outs = []
        for r in range(nslab):
            outs.append(
                pltpu.async_copy(
                    rows_v.at[pl.ds(r * rows_per_slab, rows_per_slab)],
                    out_hbm.at[r, pl.ds(wid * rows_per_slab, rows_per_slab)],
                    osem,
                )
            )
        for c in outs:
            c.wait()

    return gather_kernel


@functools.lru_cache(maxsize=None)
def _make_finish(B, S, D):
    assert 2 * D == 128 and S % 2 == 0
    R = S * D // 128

    def body(x_ref, h2_ref):
        t = jnp.transpose(x_ref[0], (1, 0))
        h2_ref[...] = t.reshape(2, D, B)

    return pl.pallas_call(
        body,
        grid=(R,),
        in_specs=[pl.BlockSpec((1, B, 128), lambda r: (r, 0, 0))],
        out_specs=pl.BlockSpec((2, D, B), lambda r: (r, 0, 0)),
        out_shape=jax.ShapeDtypeStruct((S, D, B), jnp.float32),
    )


def kernel(input_ids, table):
    bsz, seq = input_ids.shape
    vocab, dim = table.shape
    total = bsz * seq
    nslab = seq // 2
    flat = input_ids.reshape(-1).astype(jnp.int32)
    staged = _make_gather(vocab, dim, total, seq)(flat, table)
    x = staged.reshape(nslab, bsz, 2 * dim)
    h2 = _make_finish(bsz, seq, dim)(x)
    h = jnp.transpose(h2, (2, 0, 1))
    logits = jnp.zeros((bsz, seq, vocab), dtype=h.dtype)
    return (h, logits)


# speedup vs baseline: 1.1367x; 1.1367x over previous
import functools

import jax
import jax.numpy as jnp
from jax import lax
from jax.experimental import pallas as pl
from jax.experimental.pallas import tpu as pltpu
from jax.experimental.pallas import tpu_sc as plsc

_INFO = plsc.get_sparse_core_info()
_NC, _NS = _INFO.num_cores, _INFO.num_subcores
_NW = _NC * _NS


@functools.lru_cache(maxsize=None)
def _make_gather(V, D, B, S):
    assert B % _NW == 0
    b_per_w = B // _NW
    ch = 80
    assert b_per_w % ch == 0
    n_ch = b_per_w // ch
    nslab = S // 2
    rows_per_slab = b_per_w // nslab
    nb_w = b_per_w // S
    mesh = plsc.VectorSubcoreMesh(core_axis_name="c", subcore_axis_name="s")

    @functools.partial(
        pl.kernel,
        mesh=mesh,
        compiler_params=pltpu.CompilerParams(
            use_tc_tiling_on_sc=False, needs_layout_passes=False
        ),
        out_type=jax.ShapeDtypeStruct((nslab, _NW * rows_per_slab, D), jnp.float32),
        scratch_types=[
            pltpu.VMEM((b_per_w,), jnp.int32),
            pltpu.VMEM((b_per_w,), jnp.int32),
            pltpu.VMEM((b_per_w, D), jnp.float32),
            pltpu.SemaphoreType.DMA,
            pltpu.SemaphoreType.DMA,
        ],
    )
    def gather_kernel(idx_hbm, table_hbm, out_hbm, idx_v, idx_p, rows_v, sem, osem):
        wid = lax.axis_index("s") * _NC + lax.axis_index("c")
        base = wid * b_per_w
        pltpu.sync_copy(idx_hbm.at[pl.ds(base, b_per_w)], idx_v)
        lane = lax.iota(jnp.int32, 16)
        for j in range(b_per_w // 16):
            r = (16 * j) // rows_per_slab
            rem = (16 * j - r * rows_per_slab) + lane
            b_l = rem >> 1
            sp = rem & 1
            q = b_l * S + 2 * r + sp
            idx_p[pl.ds(j * 16, 16)] = plsc.load_gather(idx_v, [q])
        gathers = []
        for j in range(n_ch):
            gathers.append(
                pltpu.async_copy(
                    table_hbm.at[idx_p.at[pl.ds(j * ch, ch)]],
                    rows_v.at[pl.ds(j * ch, ch)],
                    sem,
                )
            )
        for c in gathers:
            c.wait()
        outs = []
        for r in range(nslab):
            outs.append(
                pltpu.async_copy(
                    rows_v.at[pl.ds(r * rows_per_slab, rows_per_slab)],
                    out_hbm.at[r, pl.ds(wid * rows_per_slab, rows_per_slab)],
                    osem,
                )
            )
        for c in outs:
            c.wait()

    return gather_kernel


@functools.lru_cache(maxsize=None)
def _make_finish(B, S, D, V):
    assert 2 * D == 128 and S % 2 == 0
    R = S * D // 128

    def body(x_ref, h2_ref, z_ref, zbuf, zsem):
        r = pl.program_id(0)

        @pl.when(r == 0)
        def _fill():
            zbuf[...] = jnp.zeros((2, V, B), jnp.float32)

        pltpu.make_async_copy(zbuf, z_ref.at[pl.ds(2 * r, 2)], zsem).start()
        t = jnp.transpose(x_ref[0], (1, 0))
        h2_ref[...] = t.reshape(2, D, B)

        @pl.when(r == R - 1)
        def _drain():
            for _ in range(R):
                pltpu.make_async_copy(zbuf, z_ref.at[pl.ds(0, 2)], zsem).wait()

    return pl.pallas_call(
        body,
        grid=(R,),
        in_specs=[pl.BlockSpec((1, B, 128), lambda r: (r, 0, 0))],
        out_specs=[
            pl.BlockSpec((2, D, B), lambda r: (r, 0, 0)),
            pl.BlockSpec(memory_space=pl.ANY),
        ],
        out_shape=[
            jax.ShapeDtypeStruct((S, D, B), jnp.float32),
            jax.ShapeDtypeStruct((S, V, B), jnp.float32),
        ],
        scratch_shapes=[
            pltpu.VMEM((2, V, B), jnp.float32),
            pltpu.SemaphoreType.DMA,
        ],
    )


def kernel(input_ids, table):
    bsz, seq = input_ids.shape
    vocab, dim = table.shape
    total = bsz * seq
    nslab = seq // 2
    flat = input_ids.reshape(-1).astype(jnp.int32)
    staged = _make_gather(vocab, dim, total, seq)(flat, table)
    x = staged.reshape(nslab, bsz, 2 * dim)
    h2, z = _make_finish(bsz, seq, dim, vocab)(x)
    h = jnp.transpose(h2, (2, 0, 1))
    logits = jnp.transpose(z, (2, 0, 1))
    return (h, logits)


# speedup vs baseline: 1.1438x; 1.0062x over previous
import functools

import jax
import jax.numpy as jnp
from jax import lax
from jax.experimental import pallas as pl
from jax.experimental.pallas import tpu as pltpu
from jax.experimental.pallas import tpu_sc as plsc

_INFO = plsc.get_sparse_core_info()
_NC, _NS = _INFO.num_cores, _INFO.num_subcores
_NW = _NC * _NS


@functools.lru_cache(maxsize=None)
def _make_gather(V, D, B, S):
    assert B % _NW == 0
    b_per_w = B // _NW
    ch = 80
    assert b_per_w % ch == 0
    n_ch = b_per_w // ch
    nslab = S // 2
    rows_per_slab = b_per_w // nslab
    nb_w = b_per_w // S
    mesh = plsc.VectorSubcoreMesh(core_axis_name="c", subcore_axis_name="s")

    @functools.partial(
        pl.kernel,
        mesh=mesh,
        compiler_params=pltpu.CompilerParams(
            use_tc_tiling_on_sc=False, needs_layout_passes=False
        ),
        out_type=jax.ShapeDtypeStruct((nslab, _NW * rows_per_slab, D), jnp.float32),
        scratch_types=[
            pltpu.VMEM((b_per_w,), jnp.int32),
            pltpu.VMEM((b_per_w,), jnp.int32),
            pltpu.VMEM((b_per_w, D), jnp.float32),
            pltpu.SemaphoreType.DMA,
            pltpu.SemaphoreType.DMA,
        ],
    )
    def gather_kernel(idx_hbm, table_hbm, out_hbm, idx_v, idx_p, rows_v, sem, osem):
        wid = lax.axis_index("s") * _NC + lax.axis_index("c")
        base = wid * b_per_w
        pltpu.sync_copy(idx_hbm.at[pl.ds(base, b_per_w)], idx_v)
        lane = lax.iota(jnp.int32, 16)
        gathers = []
        per_ch = ch // 16
        for j in range(n_ch):
            for jj in range(per_ch):
                k = j * per_ch + jj
                r = (16 * k) // rows_per_slab
                rem = (16 * k - r * rows_per_slab) + lane
                q = (rem >> 1) * S + 2 * r + (rem & 1)
                idx_p[pl.ds(k * 16, 16)] = plsc.load_gather(idx_v, [q])
            gathers.append(
                pltpu.async_copy(
                    table_hbm.at[idx_p.at[pl.ds(j * ch, ch)]],
                    rows_v.at[pl.ds(j * ch, ch)],
                    sem,
                )
            )
        for c in gathers:
            c.wait()
        outs = []
        for r in range(nslab):
            outs.append(
                pltpu.async_copy(
                    rows_v.at[pl.ds(r * rows_per_slab, rows_per_slab)],
                    out_hbm.at[r, pl.ds(wid * rows_per_slab, rows_per_slab)],
                    osem,
                )
            )
        for c in outs:
            c.wait()

    return gather_kernel


@functools.lru_cache(maxsize=None)
def _make_finish(B, S, D, V):
    assert 2 * D == 128 and S % 2 == 0
    R = S * D // 128

    def body(x_ref, h2_ref, z_ref, zbuf, zsem):
        r = pl.program_id(0)

        @pl.when(r == 0)
        def _fill():
            zbuf[...] = jnp.zeros((2, V, B), jnp.float32)

        pltpu.make_async_copy(zbuf, z_ref.at[pl.ds(2 * r, 2)], zsem).start()
        t = jnp.transpose(x_ref[0], (1, 0))
        h2_ref[...] = t.reshape(2, D, B)

        @pl.when(r == R - 1)
        def _drain():
            for _ in range(R):
                pltpu.make_async_copy(zbuf, z_ref.at[pl.ds(0, 2)], zsem).wait()

    return pl.pallas_call(
        body,
        grid=(R,),
        in_specs=[pl.BlockSpec((1, B, 128), lambda r: (r, 0, 0))],
        out_specs=[
            pl.BlockSpec((2, D, B), lambda r: (r, 0, 0)),
            pl.BlockSpec(memory_space=pl.ANY),
        ],
        out_shape=[
            jax.ShapeDtypeStruct((S, D, B), jnp.float32),
            jax.ShapeDtypeStruct((S, V, B), jnp.float32),
        ],
        scratch_shapes=[
            pltpu.VMEM((2, V, B), jnp.float32),
            pltpu.SemaphoreType.DMA,
        ],
    )


def kernel(input_ids, table):
    bsz, seq = input_ids.shape
    vocab, dim = table.shape
    total = bsz * seq
    nslab = seq // 2
    flat = input_ids.reshape(-1).astype(jnp.int32)
    staged = _make_gather(vocab, dim, total, seq)(flat, table)
    x = staged.reshape(nslab, bsz, 2 * dim)
    h2, z = _make_finish(bsz, seq, dim, vocab)(x)
    h = jnp.transpose(h2, (2, 0, 1))
    logits = jnp.transpose(z, (2, 0, 1))
    return (h, logits)


# speedup vs baseline: 1.1530x; 1.0081x over previous
import functools

import jax
import jax.numpy as jnp
from jax import lax
from jax.experimental import pallas as pl
from jax.experimental.pallas import tpu as pltpu
from jax.experimental.pallas import tpu_sc as plsc

_INFO = plsc.get_sparse_core_info()
_NC, _NS = _INFO.num_cores, _INFO.num_subcores
_NW = _NC * _NS


@functools.lru_cache(maxsize=None)
def _make_gather(V, D, B, S):
    assert B % _NW == 0
    b_per_w = B // _NW
    ch = 80
    assert b_per_w % ch == 0
    n_ch = b_per_w // ch
    nslab = S // 2
    rows_per_slab = b_per_w // nslab
    nb_w = b_per_w // S
    mesh = plsc.VectorSubcoreMesh(core_axis_name="c", subcore_axis_name="s")

    @functools.partial(
        pl.kernel,
        mesh=mesh,
        compiler_params=pltpu.CompilerParams(
            use_tc_tiling_on_sc=False, needs_layout_passes=False
        ),
        out_type=jax.ShapeDtypeStruct((nslab, _NW * rows_per_slab, D), jnp.float32),
        scratch_types=[
            pltpu.VMEM((b_per_w,), jnp.int32),
            pltpu.VMEM((b_per_w,), jnp.int32),
            pltpu.VMEM((b_per_w, D), jnp.float32),
            pltpu.SemaphoreType.DMA,
            pltpu.SemaphoreType.DMA,
        ],
    )
    def gather_kernel(idx_hbm, table_hbm, out_hbm, idx_v, idx_p, rows_v, sem, osem):
        wid = lax.axis_index("s") * _NC + lax.axis_index("c")
        base = wid * b_per_w
        pltpu.sync_copy(idx_hbm.at[pl.ds(base, b_per_w)], idx_v)
        lane = lax.iota(jnp.int32, 16)
        gathers = []
        per_ch = ch // 16
        for j in range(n_ch):
            for jj in range(per_ch):
                k = j * per_ch + jj
                r = (16 * k) // rows_per_slab
                rem = (16 * k - r * rows_per_slab) + lane
                q = (rem >> 1) * S + 2 * r + (rem & 1)
                idx_p[pl.ds(k * 16, 16)] = plsc.load_gather(idx_v, [q])
            gathers.append(
                pltpu.async_copy(
                    table_hbm.at[idx_p.at[pl.ds(j * ch, ch)]],
                    rows_v.at[pl.ds(j * ch, ch)],
                    sem,
                )
            )
        for c in gathers:
            c.wait()
        outs = []
        for r in range(nslab):
            outs.append(
                pltpu.async_copy(
                    rows_v.at[pl.ds(r * rows_per_slab, rows_per_slab)],
                    out_hbm.at[r, pl.ds(wid * rows_per_slab, rows_per_slab)],
                    osem,
                )
            )
        for c in outs:
            c.wait()

    return gather_kernel


@functools.lru_cache(maxsize=None)
def _make_finish(B, S, D, V):
    assert 2 * D == 128 and S % 2 == 0
    R = S * D // 128

    def body(x_ref, h2_ref, z_ref, zbuf, zsem):
        r = pl.program_id(0)

        @pl.when(r == 0)
        def _fill():
            zbuf[...] = jnp.zeros((1, V, B), jnp.float32)

        pltpu.make_async_copy(zbuf, z_ref.at[pl.ds(2 * r, 1)], zsem).start()
        pltpu.make_async_copy(zbuf, z_ref.at[pl.ds(2 * r + 1, 1)], zsem).start()
        t = jnp.transpose(x_ref[0], (1, 0))
        h2_ref[...] = t.reshape(2, D, B)

        @pl.when(r == R - 1)
        def _drain():
            for _ in range(2 * R):
                pltpu.make_async_copy(zbuf, z_ref.at[pl.ds(0, 1)], zsem).wait()

    return pl.pallas_call(
        body,
        grid=(R,),
        in_specs=[pl.BlockSpec((1, B, 128), lambda r: (r, 0, 0))],
        out_specs=[
            pl.BlockSpec((2, D, B), lambda r: (r, 0, 0)),
            pl.BlockSpec(memory_space=pl.ANY),
        ],
        out_shape=[
            jax.ShapeDtypeStruct((S, D, B), jnp.float32),
            jax.ShapeDtypeStruct((S, V, B), jnp.float32),
        ],
        scratch_shapes=[
            pltpu.VMEM((1, V, B), jnp.float32),
            pltpu.SemaphoreType.DMA,
        ],
    )


def kernel(input_ids, table):
    bsz, seq = input_ids.shape
    vocab, dim = table.shape
    total = bsz * seq
    nslab = seq // 2
    flat = input_ids.reshape(-1).astype(jnp.int32)
    staged = _make_gather(vocab, dim, total, seq)(flat, table)
    x = staged.reshape(nslab, bsz, 2 * dim)
    h2, z = _make_finish(bsz, seq, dim, vocab)(x)
    h = jnp.transpose(h2, (2, 0, 1))
    logits = jnp.transpose(z, (2, 0, 1))
    return (h, logits)
